# baseline (speedup 1.0000x reference)
import jax
import jax.numpy as jnp
from jax import lax
from jax.experimental import pallas as pl
from jax.experimental.pallas import tpu as pltpu
from jax.experimental.pallas import tpu_sc as plsc

_B = 16384
_E = 210
_NC = 2
_NS = 16
_NW = _NC * _NS
_SW = 2 * 128 * 128
_CW = _SW // 2
_HI = 222.0


def _p1_body(zp_hbm, zy_hbm, tab_hbm, rd_hbm, znir_hbm,
             tab_v, bp0, bp1, by0, by1, rd_v, tmp_v,
             sp0, sp1, sy0, sy1):
    w = lax.axis_index("s") * _NC + lax.axis_index("c")
    pltpu.sync_copy(tab_hbm, tab_v)
    zeros = jnp.zeros((16,), jnp.float32)

    lo = jnp.where(w < 18, 7 * w, 6 * w + 18)
    ns = jnp.where(w < 18, 7, 6)
    base = lo * _SW

    pltpu.make_async_copy(zp_hbm.at[pl.ds(base, _CW)], bp0, sp0).start()
    pltpu.make_async_copy(zy_hbm.at[pl.ds(base, _CW)], by0, sy0).start()
    pltpu.make_async_copy(zp_hbm.at[pl.ds(base + _CW, _CW)], bp1, sp1).start()
    pltpu.make_async_copy(zy_hbm.at[pl.ds(base + _CW, _CW)], by1, sy1).start()

    def compute_chunk(h, bp, by, zn, ir, first):
        def bt_body(bt2, carry):
            zn, ir = carry
            offb = bt2 * 256
            r0 = h * 8192 + bt2 * 128
            for g in range(8):
                o = offb + g * 16
                x = bp[pl.ds(o, 16)]
                a = bp[pl.ds(o + 128, 16)]
                ti = by[pl.ds(o, 16)]
                ta = by[pl.ds(o + 128, 16)]
                p = jnp.exp(x * -8.0)
                q = jnp.exp(a * -8.0)
                s = p + q
                m = p * q
                zn = zn + (s - 2.0 * m) / (1.0 + (s + m))
                ir = ir + jnp.maximum(x - _HI, 0.0)
                valid = (x > 0.0) & (x <= _HI)
                xi = jnp.where(valid, x, 0.0).astype(jnp.int32)
                delta = (plsc.load_gather(tab_v, [ti.astype(jnp.int32)]) * ta
                         - plsc.load_gather(tab_v, [xi]) * a)
                if first:
                    rd_v[pl.ds(r0 + g * 16, 16)] = delta
                else:
                    plsc.addupdate(rd_v.at[pl.ds(r0 + g * 16, 16)], delta)
            return zn, ir

        return plsc.parallel_loop(0, 64, 1, unroll=4, carry=(zn, ir))(bt_body)

    def pair_iter(c0, zn, ir, more, first):
        pltpu.make_async_copy(zp_hbm.at[pl.ds(0, _CW)], bp0, sp0).wait()
        pltpu.make_async_copy(zy_hbm.at[pl.ds(0, _CW)], by0, sy0).wait()
        zn, ir = compute_chunk(0, bp0, by0, zn, ir, first)

        @pl.when(more)
        def _():
            off = base + (c0 + 2) * _CW
            pltpu.make_async_copy(zp_hbm.at[pl.ds(off, _CW)], bp0, sp0).start()
            pltpu.make_async_copy(zy_hbm.at[pl.ds(off, _CW)], by0, sy0).start()

        pltpu.make_async_copy(zp_hbm.at[pl.ds(0, _CW)], bp1, sp1).wait()
        pltpu.make_async_copy(zy_hbm.at[pl.ds(0, _CW)], by1, sy1).wait()
        zn, ir = compute_chunk(1, bp1, by1, zn, ir, first)

        @pl.when(more)
        def _():
            off = base + (c0 + 3) * _CW
            pltpu.make_async_copy(zp_hbm.at[pl.ds(off, _CW)], bp1, sp1).start()
            pltpu.make_async_copy(zy_hbm.at[pl.ds(off, _CW)], by1, sy1).start()

        return zn, ir

    zn, ir = pair_iter(0, zeros, zeros, 1 < ns, True)

    def pair_body(j, carry):
        zn, ir = carry
        return pair_iter(2 * j, zn, ir, j + 1 < ns, False)

    zn, ir = lax.fori_loop(1, ns, pair_body, (zn, ir))

    lanes = lax.iota(jnp.int32, 16)
    znS = 2.0 * jnp.sum(zn)
    irS = jnp.sum(ir)
    tmp_v[...] = jnp.where(lanes == 0, znS, jnp.where(lanes == 1, irS, 0.0))
    pltpu.sync_copy(tmp_v, znir_hbm.at[w])
    pltpu.sync_copy(rd_v, rd_hbm.at[w])


def _p2_body(rd_ref, znir_ref, o_ref):
    s = jnp.sum(rd_ref[...], axis=0, keepdims=True)
    val = jnp.sum(s * s) + jnp.sum(znir_ref[...])
    o_ref[...] = jnp.reshape(val, (1, 1))


def kernel(y_pred, y, data):
    zp = y_pred.reshape(128, 128, 7, 3, 10, 2).transpose(2, 3, 4, 0, 5, 1).reshape(-1)
    zy = y.reshape(128, 128, 7, 3, 10, 2).transpose(2, 3, 4, 0, 5, 1).reshape(-1)
    tab = jnp.concatenate([data[:, 0], jnp.zeros((1,), jnp.float32)]) * (1.0 / 700.0)

    p1 = pl.kernel(
        _p1_body,
        out_type=(
            jax.ShapeDtypeStruct((_NW, _B), jnp.float32),
            jax.ShapeDtypeStruct((_NW, 16), jnp.float32),
        ),
        mesh=plsc.VectorSubcoreMesh(core_axis_name="c", subcore_axis_name="s"),
        compiler_params=pltpu.CompilerParams(needs_layout_passes=False),
        scratch_types=[
            pltpu.VMEM((224,), jnp.float32),
            pltpu.VMEM((_CW,), jnp.float32),
            pltpu.VMEM((_CW,), jnp.float32),
            pltpu.VMEM((_CW,), jnp.float32),
            pltpu.VMEM((_CW,), jnp.float32),
            pltpu.VMEM((_B,), jnp.float32),
            pltpu.VMEM((16,), jnp.float32),
            pltpu.SemaphoreType.DMA,
            pltpu.SemaphoreType.DMA,
            pltpu.SemaphoreType.DMA,
            pltpu.SemaphoreType.DMA,
        ],
    )
    rd, znir = p1(zp, zy, tab)

    out2 = pl.pallas_call(
        _p2_body,
        out_shape=jax.ShapeDtypeStruct((1, 1), jnp.float32),
    )(rd, znir)
    return out2[0, 0] / _B


# speedup vs baseline: 1.6611x; 1.6611x over previous
import jax
import jax.numpy as jnp
from jax import lax
from jax.experimental import pallas as pl
from jax.experimental.pallas import tpu as pltpu
from jax.experimental.pallas import tpu_sc as plsc

_B = 16384
_E = 210
_NC = 2
_NS = 16
_NW = _NC * _NS
_SW = 2 * 128 * 128
_CW = _SW // 2
_HI = 222.0


def _p1_body(zp_hbm, zy_hbm, tab_hbm, rd_hbm, znir_hbm,
             tab_v, bp0, bp1, by0, by1, rd_v, tmp_v,
             sp0, sp1, sy0, sy1):
    w = lax.axis_index("s") * _NC + lax.axis_index("c")
    pltpu.sync_copy(tab_hbm, tab_v)
    zeros = jnp.zeros((16,), jnp.float32)

    lo = jnp.where(w < 18, 7 * w, 6 * w + 18)
    ns = jnp.where(w < 18, 7, 6)
    base = lo * _SW

    pltpu.make_async_copy(zp_hbm.at[pl.ds(base, _CW)], bp0, sp0).start()
    pltpu.make_async_copy(zy_hbm.at[pl.ds(base, _CW)], by0, sy0).start()
    pltpu.make_async_copy(zp_hbm.at[pl.ds(base + _CW, _CW)], bp1, sp1).start()
    pltpu.make_async_copy(zy_hbm.at[pl.ds(base + _CW, _CW)], by1, sy1).start()

    def compute_chunk(h, bp, by, zn, ir, first):
        def bt_body(bt2, carry):
            zn, ir = carry
            offb = bt2 * 256
            r0 = h * 8192 + bt2 * 128
            for g in range(8):
                o = offb + g * 16
                x = bp[pl.ds(o, 16)]
                a = bp[pl.ds(o + 128, 16)]
                ti = by[pl.ds(o, 16)]
                ta = by[pl.ds(o + 128, 16)]
                p = jnp.exp(x * -8.0)
                q = jnp.exp(a * -8.0)
                s = p + q
                m = p * q
                zn = zn + (s - 2.0 * m) / (1.0 + (s + m))
                ir = ir + jnp.maximum(x - _HI, 0.0)
                valid = (x > 0.0) & (x <= _HI)
                xi = jnp.where(valid, x, 0.0).astype(jnp.int32)
                delta = (plsc.load_gather(tab_v, [ti.astype(jnp.int32)]) * ta
                         - plsc.load_gather(tab_v, [xi]) * a)
                if first:
                    rd_v[pl.ds(r0 + g * 16, 16)] = delta
                else:
                    plsc.addupdate(rd_v.at[pl.ds(r0 + g * 16, 16)], delta)
            return zn, ir

        return plsc.parallel_loop(0, 64, 1, unroll=2, carry=(zn, ir))(bt_body)

    def pair_iter(c0, zn, ir, more, first):
        pltpu.make_async_copy(zp_hbm.at[pl.ds(0, _CW)], bp0, sp0).wait()
        pltpu.make_async_copy(zy_hbm.at[pl.ds(0, _CW)], by0, sy0).wait()
        zn, ir = compute_chunk(0, bp0, by0, zn, ir, first)

        @pl.when(more)
        def _():
            off = base + (c0 + 2) * _CW
            pltpu.make_async_copy(zp_hbm.at[pl.ds(off, _CW)], bp0, sp0).start()
            pltpu.make_async_copy(zy_hbm.at[pl.ds(off, _CW)], by0, sy0).start()

        pltpu.make_async_copy(zp_hbm.at[pl.ds(0, _CW)], bp1, sp1).wait()
        pltpu.make_async_copy(zy_hbm.at[pl.ds(0, _CW)], by1, sy1).wait()
        zn, ir = compute_chunk(1, bp1, by1, zn, ir, first)

        @pl.when(more)
        def _():
            off = base + (c0 + 3) * _CW
            pltpu.make_async_copy(zp_hbm.at[pl.ds(off, _CW)], bp1, sp1).start()
            pltpu.make_async_copy(zy_hbm.at[pl.ds(off, _CW)], by1, sy1).start()

        return zn, ir

    zn, ir = pair_iter(0, zeros, zeros, 1 < ns, True)

    def pair_body(j, carry):
        zn, ir = carry
        return pair_iter(2 * j, zn, ir, j + 1 < ns, False)

    zn, ir = lax.fori_loop(1, ns, pair_body, (zn, ir))

    lanes = lax.iota(jnp.int32, 16)
    znS = 2.0 * jnp.sum(zn)
    irS = jnp.sum(ir)
    tmp_v[...] = jnp.where(lanes == 0, znS, jnp.where(lanes == 1, irS, 0.0))
    pltpu.sync_copy(tmp_v, znir_hbm.at[w])
    pltpu.sync_copy(rd_v, rd_hbm.at[w])


def _p2_body(rd_ref, znir_ref, o_ref):
    s = jnp.sum(rd_ref[...], axis=0, keepdims=True)
    val = jnp.sum(s * s) + jnp.sum(znir_ref[...])
    o_ref[...] = jnp.reshape(val, (1, 1))


def kernel(y_pred, y, data):
    zp = y_pred.reshape(128, 128, 7, 3, 10, 2).transpose(2, 3, 4, 0, 5, 1).reshape(-1)
    zy = y.reshape(128, 128, 7, 3, 10, 2).transpose(2, 3, 4, 0, 5, 1).reshape(-1)
    tab = jnp.concatenate([data[:, 0], jnp.zeros((1,), jnp.float32)]) * (1.0 / 700.0)

    p1 = pl.kernel(
        _p1_body,
        out_type=(
            jax.ShapeDtypeStruct((_NW, _B), jnp.float32),
            jax.ShapeDtypeStruct((_NW, 16), jnp.float32),
        ),
        mesh=plsc.VectorSubcoreMesh(core_axis_name="c", subcore_axis_name="s"),
        compiler_params=pltpu.CompilerParams(needs_layout_passes=False),
        scratch_types=[
            pltpu.VMEM((224,), jnp.float32),
            pltpu.VMEM((_CW,), jnp.float32),
            pltpu.VMEM((_CW,), jnp.float32),
            pltpu.VMEM((_CW,), jnp.float32),
            pltpu.VMEM((_CW,), jnp.float32),
            pltpu.VMEM((_B,), jnp.float32),
            pltpu.VMEM((16,), jnp.float32),
            pltpu.SemaphoreType.DMA,
            pltpu.SemaphoreType.DMA,
            pltpu.SemaphoreType.DMA,
            pltpu.SemaphoreType.DMA,
        ],
    )
    rd, znir = p1(zp, zy, tab)

    out2 = pl.pallas_call(
        _p2_body,
        out_shape=jax.ShapeDtypeStruct((1, 1), jnp.float32),
    )(rd, znir)
    return out2[0, 0] / _B


# speedup vs baseline: 1.6796x; 1.0111x over previous
import jax
import jax.numpy as jnp
from jax import lax
from jax.experimental import pallas as pl
from jax.experimental.pallas import tpu as pltpu
from jax.experimental.pallas import tpu_sc as plsc

_B = 16384
_E = 210
_NC = 2
_NS = 16
_NW = _NC * _NS
_SW = 2 * 128 * 128
_CW = _SW // 2
_HI = 222.0
_ROWS = 2 * _E * 128
_GRID = 10
_BR = _ROWS // _GRID


def _p1_body(zp_hbm, zy_hbm, tab_hbm, rd_hbm,
             tab_v, bp0, bp1, by0, by1, rd_v,
             sp0, sp1, sy0, sy1):
    w = lax.axis_index("s") * _NC + lax.axis_index("c")
    pltpu.sync_copy(tab_hbm, tab_v)

    lo = jnp.where(w < 18, 7 * w, 6 * w + 18)
    ns = jnp.where(w < 18, 7, 6)
    base = lo * _SW

    pltpu.make_async_copy(zp_hbm.at[pl.ds(base, _CW)], bp0, sp0).start()
    pltpu.make_async_copy(zy_hbm.at[pl.ds(base, _CW)], by0, sy0).start()
    pltpu.make_async_copy(zp_hbm.at[pl.ds(base + _CW, _CW)], bp1, sp1).start()
    pltpu.make_async_copy(zy_hbm.at[pl.ds(base + _CW, _CW)], by1, sy1).start()

    def compute_chunk(h, bp, by, first):
        def bt_body(bt2):
            offb = bt2 * 256
            r0 = h * 8192 + bt2 * 128
            for g in range(8):
                o = offb + g * 16
                x = bp[pl.ds(o, 16)]
                a = bp[pl.ds(o + 128, 16)]
                ti = by[pl.ds(o, 16)]
                ta = by[pl.ds(o + 128, 16)]
                valid = (x > 0.0) & (x <= _HI)
                xi = jnp.where(valid, x, 0.0).astype(jnp.int32)
                delta = (plsc.load_gather(tab_v, [ti.astype(jnp.int32)]) * ta
                         - plsc.load_gather(tab_v, [xi]) * a)
                if first:
                    rd_v[pl.ds(r0 + g * 16, 16)] = delta
                else:
                    plsc.addupdate(rd_v.at[pl.ds(r0 + g * 16, 16)], delta)

        plsc.parallel_loop(0, 64, 1, unroll=2)(bt_body)

    def pair_iter(c0, more, first):
        pltpu.make_async_copy(zp_hbm.at[pl.ds(0, _CW)], bp0, sp0).wait()
        pltpu.make_async_copy(zy_hbm.at[pl.ds(0, _CW)], by0, sy0).wait()
        compute_chunk(0, bp0, by0, first)

        @pl.when(more)
        def _():
            off = base + (c0 + 2) * _CW
            pltpu.make_async_copy(zp_hbm.at[pl.ds(off, _CW)], bp0, sp0).start()
            pltpu.make_async_copy(zy_hbm.at[pl.ds(off, _CW)], by0, sy0).start()

        pltpu.make_async_copy(zp_hbm.at[pl.ds(0, _CW)], bp1, sp1).wait()
        pltpu.make_async_copy(zy_hbm.at[pl.ds(0, _CW)], by1, sy1).wait()
        compute_chunk(1, bp1, by1, first)

        @pl.when(more)
        def _():
            off = base + (c0 + 3) * _CW
            pltpu.make_async_copy(zp_hbm.at[pl.ds(off, _CW)], bp1, sp1).start()
            pltpu.make_async_copy(zy_hbm.at[pl.ds(off, _CW)], by1, sy1).start()

    pair_iter(0, 1 < ns, True)

    def pair_body(j, carry):
        pair_iter(2 * j, j + 1 < ns, False)
        return carry

    lax.fori_loop(1, ns, pair_body, 0)

    pltpu.sync_copy(rd_v, rd_hbm.at[w])


def _mask_body(zp_ref, o_ref):
    blk = zp_ref[...]
    t = jnp.tanh(blk * 4.0)
    t2 = t.reshape(_BR // 2, 2, 128)
    x2 = blk.reshape(_BR // 2, 2, 128)[:, 0, :]
    zn = jnp.sum(t) - 2.0 * jnp.sum(t2[:, 0, :] * t2[:, 1, :])
    ir = jnp.sum(jnp.maximum(x2 - _HI, 0.0))
    lane = jax.lax.broadcasted_iota(jnp.int32, (1, 128), 1)
    part = jnp.where(lane == 0, zn, jnp.where(lane == 1, ir, 0.0))

    @pl.when(pl.program_id(0) == 0)
    def _():
        o_ref[...] = part

    @pl.when(pl.program_id(0) > 0)
    def _():
        o_ref[...] = o_ref[...] + part


def _p2_body(rd_ref, mask_ref, o_ref):
    s = jnp.sum(rd_ref[...], axis=0, keepdims=True)
    val = jnp.sum(s * s) + jnp.sum(mask_ref[...])
    o_ref[...] = jnp.reshape(val, (1, 1))


def kernel(y_pred, y, data):
    zp = y_pred.reshape(128, 128, 7, 3, 10, 2).transpose(2, 3, 4, 0, 5, 1).reshape(-1)
    zy = y.reshape(128, 128, 7, 3, 10, 2).transpose(2, 3, 4, 0, 5, 1).reshape(-1)
    tab = jnp.concatenate([data[:, 0], jnp.zeros((1,), jnp.float32)]) * (1.0 / 700.0)

    p1 = pl.kernel(
        _p1_body,
        out_type=jax.ShapeDtypeStruct((_NW, _B), jnp.float32),
        mesh=plsc.VectorSubcoreMesh(core_axis_name="c", subcore_axis_name="s"),
        compiler_params=pltpu.CompilerParams(needs_layout_passes=False),
        scratch_types=[
            pltpu.VMEM((224,), jnp.float32),
            pltpu.VMEM((_CW,), jnp.float32),
            pltpu.VMEM((_CW,), jnp.float32),
            pltpu.VMEM((_CW,), jnp.float32),
            pltpu.VMEM((_CW,), jnp.float32),
            pltpu.VMEM((_B,), jnp.float32),
            pltpu.SemaphoreType.DMA,
            pltpu.SemaphoreType.DMA,
            pltpu.SemaphoreType.DMA,
            pltpu.SemaphoreType.DMA,
        ],
    )
    rd = p1(zp, zy, tab)

    mask_part = pl.pallas_call(
        _mask_body,
        grid=(_GRID,),
        in_specs=[pl.BlockSpec((_BR, 128), lambda i: (i, 0))],
        out_specs=pl.BlockSpec((1, 128), lambda i: (0, 0)),
        out_shape=jax.ShapeDtypeStruct((1, 128), jnp.float32),
    )(zp.reshape(_ROWS, 128))

    out2 = pl.pallas_call(
        _p2_body,
        out_shape=jax.ShapeDtypeStruct((1, 1), jnp.float32),
    )(rd, mask_part)
    return out2[0, 0] / _B
